# baseline (speedup 1.0000x reference)
import functools

import jax
import jax.numpy as jnp
from jax.experimental import pallas as pl
from jax.experimental.pallas import tpu as pltpu


def _mm_kernel(x_ref, w_ref, b_ref, o_ref, *, act, out_dtype):
    x = x_ref[...].astype(jnp.bfloat16)
    w = w_ref[...].astype(jnp.bfloat16)
    acc = jnp.dot(x, w, preferred_element_type=jnp.float32)
    acc = acc + b_ref[...]
    if act:
        acc = jnp.maximum(acc, 0.0)
    o_ref[...] = acc.astype(out_dtype)


def _layer(h, w, b, *, bm, bn, act, out_dtype, interpret=False):
    M, K = h.shape
    _, N = w.shape
    n_tiles = N // bn
    m_tiles = M // bm
    body = functools.partial(_mm_kernel, act=act, out_dtype=out_dtype)
    return pl.pallas_call(
        body,
        grid=(n_tiles, m_tiles),
        in_specs=[
            pl.BlockSpec((bm, K), lambda n, m: (m, 0)),
            pl.BlockSpec((K, bn), lambda n, m: (0, n)),
            pl.BlockSpec((1, bn), lambda n, m: (0, n)),
        ],
        out_specs=pl.BlockSpec((bm, bn), lambda n, m: (m, n)),
        out_shape=jax.ShapeDtypeStruct((M, N), out_dtype),
        compiler_params=pltpu.CompilerParams(
            dimension_semantics=("arbitrary", "arbitrary"),
            vmem_limit_bytes=64 * 1024 * 1024,
        ),
        interpret=interpret,
    )(h, w, b)


def kernel(x, W1, b1, W2, b2, W3, b3, interpret=False):
    h1 = _layer(x, W1, b1.reshape(1, -1), bm=1024, bn=4096, act=True,
                out_dtype=jnp.bfloat16, interpret=interpret)
    h2 = _layer(h1, W2, b2.reshape(1, -1), bm=1024, bn=1024, act=True,
                out_dtype=jnp.bfloat16, interpret=interpret)
    out = _layer(h2, W3, b3.reshape(1, -1), bm=1024, bn=1000, act=False,
                 out_dtype=jnp.float32, interpret=interpret)
    return out
